# baseline (speedup 1.0000x reference)
import functools

import jax
import jax.numpy as jnp
from jax import lax
from jax.experimental import pallas as pl
from jax.experimental.pallas import tpu as pltpu
from jax.experimental.pallas import tpu_sc as plsc

_B = 16384
_D = 128
_NC = 2
_NS = 16
_NW = _NC * _NS
_BW = _B // _NW
_C = 128
_NCHUNK = _BW // _C
_G = _C // 16


def _body(users_hbm, items_hbm, utab_hbm, itab_hbm, out_hbm,
          uidx, iidx, ubuf, ibuf, outv, sem0, sem1):
    wid = lax.axis_index("s") * _NC + lax.axis_index("c")
    base = pl.multiple_of(wid * _BW, _BW)

    pltpu.sync_copy(users_hbm.at[pl.ds(base, _BW)], uidx)
    pltpu.sync_copy(items_hbm.at[pl.ds(base, _BW)], iidx)

    sems = (sem0, sem1)

    def start(c):
        s = c % 2
        cu = pltpu.async_copy(
            utab_hbm.at[uidx.at[pl.ds(c * _C, _C)]], ubuf.at[s], sems[s])
        ci = pltpu.async_copy(
            itab_hbm.at[iidx.at[pl.ds(c * _C, _C)]], ibuf.at[s], sems[s])
        return cu, ci

    lane = lax.iota(jnp.int32, 16)
    pend = start(0)
    for c in range(_NCHUNK):
        s = c % 2
        cu, ci = pend
        if c + 1 < _NCHUNK:
            nxt = start(c + 1)
        cu.wait()
        ci.wait()
        if c + 1 < _NCHUNK:
            pend = nxt

        ub = ubuf.at[s]
        ib = ibuf.at[s]

        def group(g, carry):
            dots = jnp.zeros((16,), jnp.float32)
            for p in range(16):
                row = g * 16 + p
                acc = ub[row, pl.ds(0, 16)] * ib[row, pl.ds(0, 16)]
                for k in range(1, 8):
                    acc = acc + ub[row, pl.ds(16 * k, 16)] * ib[row, pl.ds(16 * k, 16)]
                dots = jnp.where(lane == p, jnp.sum(acc), dots)
            res = 1.0 / (1.0 + jnp.exp(-dots))
            off = pl.multiple_of(c * _C + g * 16, 16)
            outv[pl.ds(off, 16)] = res
            return carry

        lax.fori_loop(0, _G, group, 0)

    pltpu.sync_copy(outv, out_hbm.at[pl.ds(base, _BW)])


@functools.partial(
    pl.kernel,
    out_type=jax.ShapeDtypeStruct((_B,), jnp.float32),
    mesh=plsc.VectorSubcoreMesh(
        core_axis_name="c", subcore_axis_name="s",
        num_cores=_NC, num_subcores=_NS),
    compiler_params=pltpu.CompilerParams(needs_layout_passes=False),
    scratch_types=[
        pltpu.VMEM((_BW,), jnp.int32),
        pltpu.VMEM((_BW,), jnp.int32),
        pltpu.VMEM((2, _C, _D), jnp.float32),
        pltpu.VMEM((2, _C, _D), jnp.float32),
        pltpu.VMEM((_BW,), jnp.float32),
        pltpu.SemaphoreType.DMA,
        pltpu.SemaphoreType.DMA,
    ],
)
def _ultragcn_sc(users_hbm, items_hbm, utab_hbm, itab_hbm, out_hbm,
                 uidx, iidx, ubuf, ibuf, outv, sem0, sem1):
    _body(users_hbm, items_hbm, utab_hbm, itab_hbm, out_hbm,
          uidx, iidx, ubuf, ibuf, outv, sem0, sem1)


def kernel(data, user_table, item_table):
    users = data[:, 0]
    items = data[:, 1]
    return _ultragcn_sc(users, items, user_table, item_table)


# speedup vs baseline: 1.3913x; 1.3913x over previous
import functools

import jax
import jax.numpy as jnp
from jax import lax
from jax.experimental import pallas as pl
from jax.experimental.pallas import tpu as pltpu
from jax.experimental.pallas import tpu_sc as plsc

_B = 16384
_D = 128
_NC = 2
_NS = 16
_NW = _NC * _NS
_BW = _B // _NW
_C = 128
_NCHUNK = _BW // _C
_G = _C // 16


def _body(users_hbm, items_hbm, utab_hbm, itab_hbm, out_hbm,
          uidx, iidx, ubuf, ibuf, tbuf, outv, sem0, sem1):
    wid = lax.axis_index("s") * _NC + lax.axis_index("c")
    base = pl.multiple_of(wid * _BW, _BW)

    pltpu.sync_copy(users_hbm.at[pl.ds(base, _BW)], uidx)
    pltpu.sync_copy(items_hbm.at[pl.ds(base, _BW)], iidx)

    sems = (sem0, sem1)

    def start(c):
        s = c % 2
        cu = pltpu.async_copy(
            utab_hbm.at[uidx.at[pl.ds(c * _C, _C)]], ubuf.at[s], sems[s])
        ci = pltpu.async_copy(
            itab_hbm.at[iidx.at[pl.ds(c * _C, _C)]], ibuf.at[s], sems[s])
        return cu, ci

    lane = lax.iota(jnp.int32, 16)
    col_base = lane * 16
    pend = start(0)
    for c in range(_NCHUNK):
        s = c % 2
        cu, ci = pend
        if c + 1 < _NCHUNK:
            nxt = start(c + 1)
        cu.wait()
        ci.wait()
        if c + 1 < _NCHUNK:
            pend = nxt

        ub = ubuf.at[s]
        ib = ibuf.at[s]

        def group(g, carry):
            for p in range(16):
                row = g * 16 + p
                acc = ub[row, pl.ds(0, 16)] * ib[row, pl.ds(0, 16)]
                for k in range(1, 8):
                    acc = acc + ub[row, pl.ds(16 * k, 16)] * ib[row, pl.ds(16 * k, 16)]
                tbuf[pl.ds(16 * p, 16)] = acc
            d0 = plsc.load_gather(tbuf, [col_base])
            d1 = plsc.load_gather(tbuf, [col_base + 1])
            d2 = plsc.load_gather(tbuf, [col_base + 2])
            d3 = plsc.load_gather(tbuf, [col_base + 3])
            for l in range(4, 16, 4):
                d0 = d0 + plsc.load_gather(tbuf, [col_base + l])
                d1 = d1 + plsc.load_gather(tbuf, [col_base + l + 1])
                d2 = d2 + plsc.load_gather(tbuf, [col_base + l + 2])
                d3 = d3 + plsc.load_gather(tbuf, [col_base + l + 3])
            dots = (d0 + d1) + (d2 + d3)
            res = 1.0 / (1.0 + jnp.exp(-dots))
            off = pl.multiple_of(c * _C + g * 16, 16)
            outv[pl.ds(off, 16)] = res
            return carry

        lax.fori_loop(0, _G, group, 0)

    pltpu.sync_copy(outv, out_hbm.at[pl.ds(base, _BW)])


@functools.partial(
    pl.kernel,
    out_type=jax.ShapeDtypeStruct((_B,), jnp.float32),
    mesh=plsc.VectorSubcoreMesh(
        core_axis_name="c", subcore_axis_name="s",
        num_cores=_NC, num_subcores=_NS),
    compiler_params=pltpu.CompilerParams(needs_layout_passes=False),
    scratch_types=[
        pltpu.VMEM((_BW,), jnp.int32),
        pltpu.VMEM((_BW,), jnp.int32),
        pltpu.VMEM((2, _C, _D), jnp.float32),
        pltpu.VMEM((2, _C, _D), jnp.float32),
        pltpu.VMEM((256,), jnp.float32),
        pltpu.VMEM((_BW,), jnp.float32),
        pltpu.SemaphoreType.DMA,
        pltpu.SemaphoreType.DMA,
    ],
)
def _ultragcn_sc(users_hbm, items_hbm, utab_hbm, itab_hbm, out_hbm,
                 uidx, iidx, ubuf, ibuf, tbuf, outv, sem0, sem1):
    _body(users_hbm, items_hbm, utab_hbm, itab_hbm, out_hbm,
          uidx, iidx, ubuf, ibuf, tbuf, outv, sem0, sem1)


def kernel(data, user_table, item_table):
    users = data[:, 0]
    items = data[:, 1]
    return _ultragcn_sc(users, items, user_table, item_table)


# speedup vs baseline: 1.4599x; 1.0493x over previous
import functools

import jax
import jax.numpy as jnp
from jax import lax
from jax.experimental import pallas as pl
from jax.experimental.pallas import tpu as pltpu
from jax.experimental.pallas import tpu_sc as plsc

_B = 16384
_D = 128
_NC = 2
_NS = 16
_NW = _NC * _NS
_BW = _B // _NW
_C = 128
_NCHUNK = _BW // _C
_G = _C // 16


def _body(users_hbm, items_hbm, utab_hbm, itab_hbm, out_hbm,
          uidx, iidx, ubuf, ibuf, tbuf, outv, sem0, sem1):
    wid = lax.axis_index("s") * _NC + lax.axis_index("c")
    base = pl.multiple_of(wid * _BW, _BW)

    pltpu.sync_copy(users_hbm.at[pl.ds(base, _BW)], uidx)
    pltpu.sync_copy(items_hbm.at[pl.ds(base, _BW)], iidx)

    sems = (sem0, sem1)

    def start(c):
        s = c % 2
        cu = pltpu.async_copy(
            utab_hbm.at[uidx.at[pl.ds(c * _C, _C)]], ubuf.at[s], sems[s])
        ci = pltpu.async_copy(
            itab_hbm.at[iidx.at[pl.ds(c * _C, _C)]], ibuf.at[s], sems[s])
        return cu, ci

    lane = lax.iota(jnp.int32, 16)
    col_base = lane * 16
    pend = start(0)
    for c in range(_NCHUNK):
        s = c % 2
        cu, ci = pend
        if c + 1 < _NCHUNK:
            nxt = start(c + 1)
        cu.wait()
        ci.wait()
        if c + 1 < _NCHUNK:
            pend = nxt

        ub = ubuf.at[s]
        ib = ibuf.at[s]

        def dot_row(row):
            a = ub[row, pl.ds(0, 16)] * ib[row, pl.ds(0, 16)]
            b = ub[row, pl.ds(16, 16)] * ib[row, pl.ds(16, 16)]
            for k in range(2, 8, 2):
                a = a + ub[row, pl.ds(16 * k, 16)] * ib[row, pl.ds(16 * k, 16)]
                b = b + ub[row, pl.ds(16 * (k + 1), 16)] * ib[row, pl.ds(16 * (k + 1), 16)]
            return a + b

        def group(g, carry):
            for p in range(0, 16, 4):
                acc0 = dot_row(g * 16 + p)
                acc1 = dot_row(g * 16 + p + 1)
                acc2 = dot_row(g * 16 + p + 2)
                acc3 = dot_row(g * 16 + p + 3)
                tbuf[pl.ds(16 * p, 16)] = acc0
                tbuf[pl.ds(16 * (p + 1), 16)] = acc1
                tbuf[pl.ds(16 * (p + 2), 16)] = acc2
                tbuf[pl.ds(16 * (p + 3), 16)] = acc3
            d0 = plsc.load_gather(tbuf, [col_base])
            d1 = plsc.load_gather(tbuf, [col_base + 1])
            d2 = plsc.load_gather(tbuf, [col_base + 2])
            d3 = plsc.load_gather(tbuf, [col_base + 3])
            for l in range(4, 16, 4):
                d0 = d0 + plsc.load_gather(tbuf, [col_base + l])
                d1 = d1 + plsc.load_gather(tbuf, [col_base + l + 1])
                d2 = d2 + plsc.load_gather(tbuf, [col_base + l + 2])
                d3 = d3 + plsc.load_gather(tbuf, [col_base + l + 3])
            dots = (d0 + d1) + (d2 + d3)
            res = 1.0 / (1.0 + jnp.exp(-dots))
            off = pl.multiple_of(c * _C + g * 16, 16)
            outv[pl.ds(off, 16)] = res
            return carry

        lax.fori_loop(0, _G, group, 0)

    pltpu.sync_copy(outv, out_hbm.at[pl.ds(base, _BW)])


@functools.partial(
    pl.kernel,
    out_type=jax.ShapeDtypeStruct((_B,), jnp.float32),
    mesh=plsc.VectorSubcoreMesh(
        core_axis_name="c", subcore_axis_name="s",
        num_cores=_NC, num_subcores=_NS),
    compiler_params=pltpu.CompilerParams(needs_layout_passes=False),
    scratch_types=[
        pltpu.VMEM((_BW,), jnp.int32),
        pltpu.VMEM((_BW,), jnp.int32),
        pltpu.VMEM((2, _C, _D), jnp.float32),
        pltpu.VMEM((2, _C, _D), jnp.float32),
        pltpu.VMEM((256,), jnp.float32),
        pltpu.VMEM((_BW,), jnp.float32),
        pltpu.SemaphoreType.DMA,
        pltpu.SemaphoreType.DMA,
    ],
)
def _ultragcn_sc(users_hbm, items_hbm, utab_hbm, itab_hbm, out_hbm,
                 uidx, iidx, ubuf, ibuf, tbuf, outv, sem0, sem1):
    _body(users_hbm, items_hbm, utab_hbm, itab_hbm, out_hbm,
          uidx, iidx, ubuf, ibuf, tbuf, outv, sem0, sem1)


def kernel(data, user_table, item_table):
    users = data[:, 0]
    items = data[:, 1]
    return _ultragcn_sc(users, items, user_table, item_table)
